# baseline (device time: 38121 ns/iter reference)
import jax
import jax.numpy as jnp
from jax import lax
from jax.experimental import pallas as pl
from jax.experimental.pallas import tpu as pltpu

N_DEV = 32
B, SQ, SKV, HQ_LOC, DH = 2, 256, 256, 4, 64
D_MODEL = 512
ROWS = B * SQ
PIECE = ROWS // N_DEV
HD_LOC = HQ_LOC * DH
BLK = 64
HALF = D_MODEL // 2

OFFS = sorted(range(1, N_DEV), key=lambda o: -min(o, N_DEV - o))


def kernel(x, Wq, K_ext, V_ext, Wo):
    def body(x_ref, wq_ref, k_ref, v_ref, wo_ref,
             out_ref,
             wq_loc, wo_loc, partial, rs_buf, g_buf,
             local_sems, send1, recv1, send2, recv2):
        me = lax.axis_index("i")

        wq_cp = pltpu.make_async_copy(
            wq_ref.at[:, pl.ds(me * HD_LOC, HD_LOC)], wq_loc,
            local_sems.at[0])
        wo_cp = pltpu.make_async_copy(
            wo_ref.at[pl.ds(me * HD_LOC, HD_LOC), :], wo_loc,
            local_sems.at[1])
        wq_cp.start()
        wo_cp.start()

        barrier = pltpu.get_barrier_semaphore()
        for off in OFFS:
            pl.semaphore_signal(
                barrier, inc=1, device_id=((me + off) % N_DEV,),
                device_id_type=pl.DeviceIdType.MESH)

        wq_cp.wait()
        x2 = x_ref[...].reshape(ROWS, D_MODEL).astype(jnp.bfloat16)
        q = jnp.dot(x2, wq_loc[...].astype(jnp.bfloat16),
                    preferred_element_type=jnp.float32) * 0.125

        row = lax.broadcasted_iota(jnp.int32, (SQ, SKV), 0)
        col = lax.broadcasted_iota(jnp.int32, (SQ, SKV), 1)
        mask = (col // BLK) <= (row // BLK)

        ctx_rows = []
        for b in range(B):
            heads = []
            for h in range(HQ_LOC):
                q_bh = q[b * SQ:(b + 1) * SQ,
                         h * DH:(h + 1) * DH].astype(jnp.bfloat16)
                k_bh = k_ref[b, :, h, :].astype(jnp.bfloat16)
                v_bh = v_ref[b, :, h, :].astype(jnp.bfloat16)
                s = lax.dot_general(
                    q_bh, k_bh, (((1,), (1,)), ((), ())),
                    preferred_element_type=jnp.float32)
                s = jnp.where(mask, s, -1e9)
                m = jnp.max(s, axis=-1, keepdims=True)
                w = jnp.exp(s - m)
                w = w / jnp.sum(w, axis=-1, keepdims=True)
                heads.append(jnp.dot(w.astype(jnp.bfloat16), v_bh,
                                     preferred_element_type=jnp.float32))
            ctx_rows.append(jnp.concatenate(heads, axis=1))
        ctx = jnp.concatenate(ctx_rows, axis=0).astype(jnp.bfloat16)

        wo_cp.wait()
        partial[...] = jnp.dot(
            ctx, wo_loc[...].astype(jnp.bfloat16),
            preferred_element_type=jnp.float32).astype(jnp.bfloat16)

        rs_buf[pl.ds(0, 1)] = partial[pl.ds(me * PIECE, PIECE), :][None]

        pl.semaphore_wait(barrier, N_DEV - 1)

        sends1 = []
        for hf in range(2):
            for off in OFFS:
                p = (me + off) % N_DEV
                rdma = pltpu.make_async_remote_copy(
                    src_ref=partial.at[pl.ds(p * PIECE, PIECE),
                                       pl.ds(hf * HALF, HALF)],
                    dst_ref=rs_buf.at[off, :, pl.ds(hf * HALF, HALF)],
                    send_sem=send1.at[hf, off],
                    recv_sem=recv1.at[hf, off],
                    device_id=(p,),
                    device_id_type=pl.DeviceIdType.MESH,
                )
                rdma.start()
                sends1.append(rdma)

        sends2 = []
        for hf in range(2):
            for off in OFFS:
                rdma = pltpu.make_async_remote_copy(
                    src_ref=partial.at[pl.ds(0, PIECE),
                                       pl.ds(hf * HALF, HALF)],
                    dst_ref=rs_buf.at[off, :, pl.ds(hf * HALF, HALF)],
                    send_sem=send1.at[hf, off],
                    recv_sem=recv1.at[hf, off],
                    device_id=(0,),
                    device_id_type=pl.DeviceIdType.MESH,
                )
                rdma.wait_recv()

            reduced = jnp.sum(
                rs_buf[:, :, hf * HALF:(hf + 1) * HALF].astype(jnp.float32),
                axis=0).astype(jnp.bfloat16)
            g_buf[pl.ds(me, 1), :, pl.ds(hf * HALF, HALF)] = reduced[None]

            for off in OFFS:
                p = (me + off) % N_DEV
                rdma = pltpu.make_async_remote_copy(
                    src_ref=g_buf.at[me, :, pl.ds(hf * HALF, HALF)],
                    dst_ref=g_buf.at[me, :, pl.ds(hf * HALF, HALF)],
                    send_sem=send2.at[hf, off],
                    recv_sem=recv2.at[hf, off],
                    device_id=(p,),
                    device_id_type=pl.DeviceIdType.MESH,
                )
                rdma.start()
                sends2.append(rdma)

        for hf in range(2):
            for off in OFFS:
                s_ = (me - off) % N_DEV
                rdma = pltpu.make_async_remote_copy(
                    src_ref=g_buf.at[0, :, pl.ds(hf * HALF, HALF)],
                    dst_ref=g_buf.at[s_, :, pl.ds(hf * HALF, HALF)],
                    send_sem=send2.at[hf, off],
                    recv_sem=recv2.at[hf, off],
                    device_id=(0,),
                    device_id_type=pl.DeviceIdType.MESH,
                )
                rdma.wait_recv()

        out_ref[...] = g_buf[...].astype(jnp.float32).reshape(B, SQ, D_MODEL)

        for rdma in sends1 + sends2:
            rdma.wait_send()

    return pl.pallas_call(
        body,
        out_shape=jax.ShapeDtypeStruct((B, SQ, D_MODEL), jnp.float32),
        in_specs=[
            pl.BlockSpec(memory_space=pltpu.VMEM),
            pl.BlockSpec(memory_space=pl.ANY),
            pl.BlockSpec(memory_space=pltpu.VMEM),
            pl.BlockSpec(memory_space=pltpu.VMEM),
            pl.BlockSpec(memory_space=pl.ANY),
        ],
        out_specs=pl.BlockSpec(memory_space=pltpu.VMEM),
        scratch_shapes=[
            pltpu.VMEM((D_MODEL, HD_LOC), jnp.float32),
            pltpu.VMEM((HD_LOC, D_MODEL), jnp.float32),
            pltpu.VMEM((ROWS, D_MODEL), jnp.bfloat16),
            pltpu.VMEM((N_DEV, PIECE, D_MODEL), jnp.bfloat16),
            pltpu.VMEM((N_DEV, PIECE, D_MODEL), jnp.bfloat16),
            pltpu.SemaphoreType.DMA((2,)),
            pltpu.SemaphoreType.DMA((2, N_DEV)),
            pltpu.SemaphoreType.DMA((2, N_DEV)),
            pltpu.SemaphoreType.DMA((2, N_DEV)),
            pltpu.SemaphoreType.DMA((2, N_DEV)),
        ],
        compiler_params=pltpu.CompilerParams(collective_id=0),
    )(x, Wq, K_ext, V_ext, Wo)
